# baseline (speedup 1.0000x reference)
import functools

import numpy as np

import jax
from jax import lax
import jax.numpy as jnp
from jax.experimental import pallas as pl
from jax.experimental.pallas import tpu as pltpu
from jax.experimental.pallas import tpu_sc as plsc

_N = 1024
_D = 32
_H = 4
_DH = 8
_KITER = 4
_DMEM = 10
_EALL = _N * (_N - 1) // 2
_LPAD = 557056
_EBLK = 8192
_TB = 256
_ST = 16
_CROWS = 32
_WINSZ = 32768


def _edge_bias_body(ys0, ys1, yc, ln, w1, b1, w2, b2, out):
    r0 = ys0[...]
    r1 = ys1[...]
    r2 = yc[...]
    lm = ln[...] > 0.5
    acc = [jnp.zeros_like(r0) for _ in range(_H)]
    for c in range(8):
        h1 = r0 * w1[0, c] + r1 * w1[1, c] + r2 * w1[2, c] + b1[c]
        h1 = jnp.where(h1 > 0, h1, 0.1 * h1)
        for h in range(_H):
            acc[h] = acc[h] + h1 * w2[c, h]
    rows = [jnp.where(lm, acc[h] + b2[h], -jnp.inf) for h in range(_H)]
    eb = out.shape[1]
    n = out.shape[2]
    out[...] = jnp.concatenate(
        [r.reshape(1, eb, n) for r in rows], axis=0)


def _sc_unflatten_body(p_hbm, out_hbm, win_v, rowbuf_v):
    n = _N
    wid = lax.axis_index("s") * 2 + lax.axis_index("c")

    def chunk_body(t, carry):
        task = wid * 4 + t
        h = task // (n // _CROWS)
        i0 = (task % (n // _CROWS)) * _CROWS
        start0 = i0 * (n - 1) - (i0 * (i0 - 1)) // 2 - i0
        base8 = (start0 // 8) * 8
        src_off = pl.multiple_of(h * _LPAD + base8, 8)
        pltpu.sync_copy(p_hbm.at[pl.ds(src_off, _WINSZ)], win_v)

        def row_body(r, c2):
            i = i0 + r
            st = i * (n - 1) - (i * (i - 1)) // 2 - i - base8
            for g in range(n // 16):
                rowbuf_v[pl.ds(r * n + g * 16, 16)] = \
                    win_v[pl.ds(st + g * 16, 16)]
            return c2

        lax.fori_loop(0, _CROWS, row_body, 0)
        dst_off = pl.multiple_of((h * n + i0) * n, 8)
        pltpu.sync_copy(rowbuf_v, out_hbm.at[pl.ds(dst_off, _CROWS * n)])
        return carry

    lax.fori_loop(0, 4, chunk_body, 0)


def _sym_body(a_ref, b_ref, out_ref):
    ib = pl.program_id(1)
    jb = pl.program_id(2)
    a = a_ref[0]
    bt = jnp.transpose(b_ref[0])
    tb = a.shape[0]
    ri = jax.lax.broadcasted_iota(jnp.int32, (tb, tb), 0) + ib * tb
    ci = jax.lax.broadcasted_iota(jnp.int32, (tb, tb), 1) + jb * tb
    t = jnp.where(ci > ri, a, 0.0) + jnp.where(ci < ri, bt, 0.0)
    out_ref[0] = jnp.where(ci == ri, -jnp.inf, t)


def _attn_body(bias_ref, st_ref, win_ref, bin_ref, g1_ref, c1_ref,
               wq_ref, wk_ref, wv_ref, wo_ref, g2_ref, c2_ref,
               wf1_ref, bf1_ref, wf2_ref, bf2_ref, wupd_ref, bupd_ref,
               out_ref):
    f32 = jnp.float32
    win = win_ref[...]
    binr = bin_ref[...]
    g1 = g1_ref[...]
    c1 = c1_ref[...]
    g2 = g2_ref[...]
    c2 = c2_ref[...]
    wf1 = wf1_ref[...]
    bf1 = bf1_ref[...]
    wf2 = wf2_ref[...]
    bf2 = bf2_ref[...]
    def k_body(k, st):
        x = jnp.dot(st, win, preferred_element_type=f32) + binr
        mu = jnp.mean(x, -1, keepdims=True)
        va = jnp.mean((x - mu) ** 2, -1, keepdims=True)
        y = (x - mu) * jax.lax.rsqrt(va + 1e-5) * g1 + c1

        def head_body(h, acc):
            qh = jnp.dot(y, wq_ref[h], preferred_element_type=f32)
            kh = jnp.dot(y, wk_ref[h], preferred_element_type=f32)
            vh = jnp.dot(y, wv_ref[h], preferred_element_type=f32)
            s = jax.lax.dot_general(qh, kh, (((1,), (1,)), ((), ())),
                                    preferred_element_type=f32)
            e = jnp.exp(s + bias_ref[h])
            den = jnp.sum(e, -1, keepdims=True)
            o = jnp.dot(e, vh, preferred_element_type=f32)
            o = o / (den + 1e-12)
            return acc + jnp.dot(o, wo_ref[h], preferred_element_type=f32)

        attn = jax.lax.fori_loop(
            0, _H, head_body, jnp.zeros((x.shape[0], _D), f32))
        x = x + attn
        mu2 = jnp.mean(x, -1, keepdims=True)
        va2 = jnp.mean((x - mu2) ** 2, -1, keepdims=True)
        z = (x - mu2) * jax.lax.rsqrt(va2 + 1e-5) * g2 + c2
        z = jax.nn.gelu(jnp.dot(z, wf1, preferred_element_type=f32) + bf1)
        z = jnp.dot(z, wf2, preferred_element_type=f32) + bf2
        x = x + z
        return st + jnp.dot(x, wupd_ref[k], preferred_element_type=f32) \
            + bupd_ref[k]

    out_ref[...] = jax.lax.fori_loop(0, _KITER, k_body, st_ref[...])


def kernel(bus_type, Line, Y, Ys, Yc, S, V0, n_nodes_per_graph, params):
    p = params
    f32 = jnp.float32

    lead = jnp.zeros((1,), f32)
    tail = jnp.zeros((_LPAD - _EALL - 1,), f32)
    ys0 = jnp.concatenate([lead, Ys[:, 0], tail])[None, :]
    ys1 = jnp.concatenate([lead, Ys[:, 1], tail])[None, :]
    yc = jnp.concatenate([lead, Yc, tail])[None, :]
    linef = jnp.concatenate([lead, Line.astype(f32), tail])[None, :]

    n_eblk = _LPAD // _EBLK
    edge_vals = pl.pallas_call(
        _edge_bias_body,
        grid=(n_eblk,),
        in_specs=[
            pl.BlockSpec((1, _EBLK), lambda i: (0, i)),
            pl.BlockSpec((1, _EBLK), lambda i: (0, i)),
            pl.BlockSpec((1, _EBLK), lambda i: (0, i)),
            pl.BlockSpec((1, _EBLK), lambda i: (0, i)),
            pl.BlockSpec(memory_space=pltpu.SMEM),
            pl.BlockSpec(memory_space=pltpu.SMEM),
            pl.BlockSpec(memory_space=pltpu.SMEM),
            pl.BlockSpec(memory_space=pltpu.SMEM),
        ],
        out_specs=pl.BlockSpec((_H, _EBLK // _N, _N), lambda i: (0, i, 0)),
        out_shape=jax.ShapeDtypeStruct((_H, _LPAD // _N, _N), f32),
    )(ys0, ys1, yc, linef, p["We1"], p["be1"], p["We2"], p["be2"])

    sc_mesh = plsc.VectorSubcoreMesh(core_axis_name="c", subcore_axis_name="s")
    u_mat = pl.kernel(
        _sc_unflatten_body,
        mesh=sc_mesh,
        out_type=jax.ShapeDtypeStruct((_H * _N * _N,), f32),
        scratch_types=[
            pltpu.VMEM((_WINSZ,), f32),
            pltpu.VMEM((_CROWS * _N,), f32),
        ],
    )(edge_vals.reshape(_H * _LPAD)).reshape(_H, _N, _N)

    bias_mat = pl.pallas_call(
        _sym_body,
        grid=(_H, _N // _TB, _N // _TB),
        in_specs=[
            pl.BlockSpec((1, _TB, _TB), lambda h, i, j: (h, i, j)),
            pl.BlockSpec((1, _TB, _TB), lambda h, i, j: (h, j, i)),
        ],
        out_specs=pl.BlockSpec((1, _TB, _TB), lambda h, i, j: (h, i, j)),
        out_shape=jax.ShapeDtypeStruct((_H, _N, _N), f32),
    )(u_mat, u_mat)

    st0 = jnp.concatenate(
        [V0[0, :, 0:1], V0[0, :, 1:2], S[0, :, 0:1], S[0, :, 1:2],
         jnp.zeros((_N, _ST - 4), f32)], axis=1)
    win16 = jnp.concatenate(
        [p["Win"], jnp.zeros((_ST - 4 - _DMEM, _D), f32)], axis=0)
    wq4 = p["Wq"].reshape(_D, _H, _DH).transpose(1, 0, 2) \
        * np.float32(1.0 / np.sqrt(_DH))
    wk4 = p["Wk"].reshape(_D, _H, _DH).transpose(1, 0, 2)
    wv4 = p["Wv"].reshape(_D, _H, _DH).transpose(1, 0, 2)
    wo4 = p["Wo"].reshape(_H, _DH, _D)
    z2 = jnp.zeros((_D, 2), f32)
    wupd = jnp.stack([
        jnp.concatenate([p["Wvh"][k][:, None], p["Wth"][k][:, None],
                         z2, p["Wm"][k], z2], axis=1)
        for k in range(_KITER)])
    bupd = jnp.stack([
        jnp.concatenate([p["bvh"][k:k + 1], p["bth"][k:k + 1],
                         jnp.zeros((2,), f32), p["bm"][k],
                         jnp.zeros((2,), f32)])[None, :]
        for k in range(_KITER)])

    st_out = pl.pallas_call(
        _attn_body,
        out_shape=jax.ShapeDtypeStruct((_N, _ST), f32),
    )(bias_mat, st0, win16, p["bin"][None, :], p["ln1_g"][None, :],
      p["ln1_b"][None, :], wq4, wk4, wv4, wo4,
      p["ln2_g"][None, :], p["ln2_b"][None, :], p["Wf1"],
      p["bf1"][None, :], p["Wf2"], p["bf2"][None, :], wupd, bupd)

    return st_out[None, :, 0:2]


# speedup vs baseline: 1.1736x; 1.1736x over previous
import numpy as np

import jax
from jax import lax
import jax.numpy as jnp
from jax.experimental import pallas as pl
from jax.experimental.pallas import tpu as pltpu
from jax.experimental.pallas import tpu_sc as plsc

_N = 1024
_D = 32
_H = 4
_DH = 8
_KITER = 4
_DMEM = 10
_EALL = _N * (_N - 1) // 2
_LPAD = 557056
_EBLK = 8192
_TB = 256
_ST = 16
_CROWS = 32
_WINSZ = 32768


def _edge_bias_body(ys0, ys1, yc, ln, w1, b1, w2, b2, out):
    r0 = ys0[...]
    r1 = ys1[...]
    r2 = yc[...]
    lm = ln[...] > 0.5
    acc = [jnp.zeros_like(r0) for _ in range(_H)]
    for c in range(8):
        h1 = r0 * w1[0, c] + r1 * w1[1, c] + r2 * w1[2, c] + b1[c]
        h1 = jnp.where(h1 > 0, h1, 0.1 * h1)
        for h in range(_H):
            acc[h] = acc[h] + h1 * w2[c, h]
    rows = [jnp.where(lm, acc[h] + b2[h], -jnp.inf) for h in range(_H)]
    eb = out.shape[1]
    n = out.shape[2]
    out[...] = jnp.concatenate(
        [r.reshape(1, eb, n) for r in rows], axis=0)


def _sc_unflatten_body(p_hbm, out_hbm, win_v, rowbuf_v):
    n = _N
    wid = lax.axis_index("s") * 2 + lax.axis_index("c")

    def chunk_body(t, carry):
        task = wid * 4 + t
        h = task // (n // _CROWS)
        i0 = (task % (n // _CROWS)) * _CROWS
        start0 = i0 * (n - 1) - (i0 * (i0 - 1)) // 2 - i0
        base8 = (start0 // 8) * 8
        src_off = pl.multiple_of(h * _LPAD + base8, 8)
        pltpu.sync_copy(p_hbm.at[pl.ds(src_off, _WINSZ)], win_v)

        def row_body(r, c2):
            i = i0 + r
            st = i * (n - 1) - (i * (i - 1)) // 2 - i - base8
            for g in range(n // 16):
                rowbuf_v[pl.ds(r * n + g * 16, 16)] = \
                    win_v[pl.ds(st + g * 16, 16)]
            return c2

        lax.fori_loop(0, _CROWS, row_body, 0)
        dst_off = pl.multiple_of((h * n + i0) * n, 8)
        pltpu.sync_copy(rowbuf_v, out_hbm.at[pl.ds(dst_off, _CROWS * n)])
        return carry

    lax.fori_loop(0, 4, chunk_body, 0)


def _attn_body(u_ref, st_ref, win_ref, bin_ref, g1_ref, c1_ref,
               wq_ref, wk_ref, wv_ref, wo_ref, g2_ref, c2_ref,
               wf1_ref, bf1_ref, wf2_ref, bf2_ref, wupd_ref, bupd_ref,
               out_ref, bias_ref):
    f32 = jnp.float32
    tbi = jax.lax.broadcasted_iota(jnp.int32, (_TB, _TB), 0)
    tbj = jax.lax.broadcasted_iota(jnp.int32, (_TB, _TB), 1)
    for h in range(_H):
        for bi in range(_N // _TB):
            for bj in range(_N // _TB):
                a = u_ref[h, bi * _TB:(bi + 1) * _TB, bj * _TB:(bj + 1) * _TB]
                bt = jnp.transpose(
                    u_ref[h, bj * _TB:(bj + 1) * _TB, bi * _TB:(bi + 1) * _TB])
                ri = tbi + bi * _TB
                ci = tbj + bj * _TB
                t = jnp.where(ci > ri, a, 0.0) + jnp.where(ci < ri, bt, 0.0)
                bias_ref[h, bi * _TB:(bi + 1) * _TB, bj * _TB:(bj + 1) * _TB] \
                    = jnp.where(ci == ri, -jnp.inf, t)
    win = win_ref[...]
    binr = bin_ref[...]
    g1 = g1_ref[...]
    c1 = c1_ref[...]
    g2 = g2_ref[...]
    c2 = c2_ref[...]
    wf1 = wf1_ref[...]
    bf1 = bf1_ref[...]
    wf2 = wf2_ref[...]
    bf2 = bf2_ref[...]
    def k_body(k, st):
        x = jnp.dot(st, win, preferred_element_type=f32) + binr
        mu = jnp.mean(x, -1, keepdims=True)
        va = jnp.mean((x - mu) ** 2, -1, keepdims=True)
        y = (x - mu) * jax.lax.rsqrt(va + 1e-5) * g1 + c1

        def head_body(h, acc):
            qh = jnp.dot(y, wq_ref[h], preferred_element_type=f32)
            kh = jnp.dot(y, wk_ref[h], preferred_element_type=f32)
            vh = jnp.dot(y, wv_ref[h], preferred_element_type=f32)
            s = jax.lax.dot_general(qh, kh, (((1,), (1,)), ((), ())),
                                    preferred_element_type=f32)
            e = jnp.exp(s + bias_ref[h])
            den = jnp.sum(e, -1, keepdims=True)
            o = jnp.dot(e, vh, preferred_element_type=f32)
            o = o / (den + 1e-12)
            return acc + jnp.dot(o, wo_ref[h], preferred_element_type=f32)

        attn = jax.lax.fori_loop(
            0, _H, head_body, jnp.zeros((x.shape[0], _D), f32))
        x = x + attn
        mu2 = jnp.mean(x, -1, keepdims=True)
        va2 = jnp.mean((x - mu2) ** 2, -1, keepdims=True)
        z = (x - mu2) * jax.lax.rsqrt(va2 + 1e-5) * g2 + c2
        z = jax.nn.gelu(jnp.dot(z, wf1, preferred_element_type=f32) + bf1)
        z = jnp.dot(z, wf2, preferred_element_type=f32) + bf2
        x = x + z
        return st + jnp.dot(x, wupd_ref[k], preferred_element_type=f32) \
            + bupd_ref[k]

    out_ref[...] = jax.lax.fori_loop(0, _KITER, k_body, st_ref[...])


def kernel(bus_type, Line, Y, Ys, Yc, S, V0, n_nodes_per_graph, params):
    p = params
    f32 = jnp.float32

    lead = jnp.zeros((1,), f32)
    tail = jnp.zeros((_LPAD - _EALL - 1,), f32)
    ys0 = jnp.concatenate([lead, Ys[:, 0], tail])[None, :]
    ys1 = jnp.concatenate([lead, Ys[:, 1], tail])[None, :]
    yc = jnp.concatenate([lead, Yc, tail])[None, :]
    linef = jnp.concatenate([lead, Line.astype(f32), tail])[None, :]

    n_eblk = _LPAD // _EBLK
    edge_vals = pl.pallas_call(
        _edge_bias_body,
        grid=(n_eblk,),
        in_specs=[
            pl.BlockSpec((1, _EBLK), lambda i: (0, i)),
            pl.BlockSpec((1, _EBLK), lambda i: (0, i)),
            pl.BlockSpec((1, _EBLK), lambda i: (0, i)),
            pl.BlockSpec((1, _EBLK), lambda i: (0, i)),
            pl.BlockSpec(memory_space=pltpu.SMEM),
            pl.BlockSpec(memory_space=pltpu.SMEM),
            pl.BlockSpec(memory_space=pltpu.SMEM),
            pl.BlockSpec(memory_space=pltpu.SMEM),
        ],
        out_specs=pl.BlockSpec((_H, _EBLK // _N, _N), lambda i: (0, i, 0)),
        out_shape=jax.ShapeDtypeStruct((_H, _LPAD // _N, _N), f32),
    )(ys0, ys1, yc, linef, p["We1"], p["be1"], p["We2"], p["be2"])

    sc_mesh = plsc.VectorSubcoreMesh(core_axis_name="c", subcore_axis_name="s")
    u_mat = pl.kernel(
        _sc_unflatten_body,
        mesh=sc_mesh,
        out_type=jax.ShapeDtypeStruct((_H * _N * _N,), f32),
        scratch_types=[
            pltpu.VMEM((_WINSZ,), f32),
            pltpu.VMEM((_CROWS * _N,), f32),
        ],
    )(edge_vals.reshape(_H * _LPAD)).reshape(_H, _N, _N)

    st0 = jnp.concatenate(
        [V0[0, :, 0:1], V0[0, :, 1:2], S[0, :, 0:1], S[0, :, 1:2],
         jnp.zeros((_N, _ST - 4), f32)], axis=1)
    win16 = jnp.concatenate(
        [p["Win"], jnp.zeros((_ST - 4 - _DMEM, _D), f32)], axis=0)
    wq4 = p["Wq"].reshape(_D, _H, _DH).transpose(1, 0, 2) \
        * np.float32(1.0 / np.sqrt(_DH))
    wk4 = p["Wk"].reshape(_D, _H, _DH).transpose(1, 0, 2)
    wv4 = p["Wv"].reshape(_D, _H, _DH).transpose(1, 0, 2)
    wo4 = p["Wo"].reshape(_H, _DH, _D)
    z2 = jnp.zeros((_D, 2), f32)
    wupd = jnp.stack([
        jnp.concatenate([p["Wvh"][k][:, None], p["Wth"][k][:, None],
                         z2, p["Wm"][k], z2], axis=1)
        for k in range(_KITER)])
    bupd = jnp.stack([
        jnp.concatenate([p["bvh"][k:k + 1], p["bth"][k:k + 1],
                         jnp.zeros((2,), f32), p["bm"][k],
                         jnp.zeros((2,), f32)])[None, :]
        for k in range(_KITER)])

    st_out = pl.pallas_call(
        _attn_body,
        out_shape=jax.ShapeDtypeStruct((_N, _ST), f32),
        scratch_shapes=[pltpu.VMEM((_H, _N, _N), f32)],
    )(u_mat, st0, win16, p["bin"][None, :], p["ln1_g"][None, :],
      p["ln1_b"][None, :], wq4, wk4, wv4, wo4,
      p["ln2_g"][None, :], p["ln2_b"][None, :], p["Wf1"],
      p["bf1"][None, :], p["Wf2"], p["bf2"][None, :], wupd, bupd)

    return st_out[None, :, 0:2]
